# baseline (speedup 1.0000x reference)
import functools

import jax
import jax.numpy as jnp
from jax import lax
from jax.experimental import pallas as pl
from jax.experimental.pallas import tpu as pltpu
from jax.experimental.pallas import tpu_sc as plsc

VOCAB = 50265
H = 768
B = 4
S = 8192
EPS = 1e-06

NC, NS, L = 2, 16, 16
NW = NC * NS
NT = B * S
TPW = NT // NW
C = 32
NCHUNK = TPW // C
NJ = H // L
G = 16
NGRP = C // G

_mesh = plsc.VectorSubcoreMesh(core_axis_name="c", subcore_axis_name="s")

_GATHER_DN = lax.GatherDimensionNumbers(
    offset_dims=(), collapsed_slice_dims=(0,), start_index_map=(0,))


def _lane_shuffle_xor(x, sft):
    idx = (lax.iota(jnp.int32, L) ^ sft).reshape(L, 1)
    return lax.gather(x, idx, _GATHER_DN, (1,),
                      mode=lax.GatherScatterMode.PROMISE_IN_BOUNDS)


@functools.partial(
    pl.kernel,
    mesh=_mesh,
    out_type=jax.ShapeDtypeStruct((NT, H), jnp.float32),
    scratch_types=[
        pltpu.VMEM((NCHUNK, C), jnp.int32),
        pltpu.VMEM((TPW,), jnp.int32),
        pltpu.VMEM((TPW + L,), jnp.float32),
        pltpu.VMEM((2, H), jnp.float32),
        pltpu.VMEM((H,), jnp.float32),
        pltpu.VMEM((H,), jnp.float32),
        pltpu.VMEM((C, H), jnp.float32),
        pltpu.VMEM((C, H), jnp.float32),
        pltpu.VMEM((C, H), jnp.float32),
        pltpu.VMEM((C, H), jnp.float32),
        pltpu.SemaphoreType.DMA,
        pltpu.SemaphoreType.DMA,
        pltpu.SemaphoreType.DMA,
        pltpu.SemaphoreType.DMA,
    ],
)
def _sc_embed_norm(ids_h, tids_h, we_h, te_h, nw_h, out_h,
                   idx_v, tid_v, tidf_v, te_v, dte_v, nw_v,
                   in0, in1, ob0, ob1, gs0, gs1, os0, os1):
    wid = lax.axis_index("s") * NC + lax.axis_index("c")
    pltpu.sync_copy(ids_h.at[wid], idx_v)
    pltpu.sync_copy(tids_h.at[wid], tid_v)
    pltpu.sync_copy(te_h, te_v)
    pltpu.sync_copy(nw_h, nw_v)

    ins = (in0, in1)
    obs = (ob0, ob1)
    gsems = (gs0, gs1)
    osems = (os0, os1)

    for j in range(NJ):
        sl = pl.ds(j * L, L)
        dte_v[sl] = te_v[1, sl] - te_v[0, sl]

    def cvt(k, _):
        sl = pl.ds(k * L, L)
        tidf_v[sl] = tid_v[sl].astype(jnp.float32)
        return 0
    lax.fori_loop(0, TPW // L, cvt, 0)

    def gather_start(c, b):
        pltpu.async_copy(we_h.at[idx_v.at[c]], ins[b], gsems[b])

    def gather_wait(c, b):
        pltpu.make_async_copy(we_h.at[idx_v.at[c]], ins[b], gsems[b]).wait()

    def out_start(c, b):
        dst = out_h.at[pl.ds(pl.multiple_of(wid * TPW + c * C, C), C)]
        pltpu.async_copy(obs[b], dst, osems[b])

    def out_wait(c, b):
        dst = out_h.at[pl.ds(pl.multiple_of(wid * TPW + c * C, C), C)]
        pltpu.make_async_copy(obs[b], dst, osems[b]).wait()

    gather_start(0, 0)
    gather_start(1, 1)

    def outer(k, _):
        for b in range(2):
            c = k * 2 + b
            ib = ins[b]
            ob = obs[b]
            gather_wait(c, b)

            @pl.when(k >= 1)
            def _():
                out_wait(c - 2, b)

            def group_body(g, _):
                gb = g * G
                tfv = tidf_v[pl.ds(c * C + gb, L)]
                tfbs = [jnp.full((L,), tfv[t]) for t in range(G)]

                def pass1(j, accs):
                    sl = pl.ds(j * L, L)
                    te0 = te_v[0, sl]
                    dte = dte_v[sl]
                    out = []
                    for t in range(G):
                        x = ib[gb + t, sl] + (te0 + tfbs[t] * dte)
                        ob[gb + t, sl] = x
                        out.append(accs[t] + x * x)
                    return tuple(out)

                zeros = tuple(jnp.zeros((L,), jnp.float32) for _ in range(G))
                accs = lax.fori_loop(0, NJ, pass1, zeros)

                regs = list(accs)
                step = 1
                lanes = lax.iota(jnp.int32, L)
                while len(regs) > 1:
                    sel = (lanes & step) != 0
                    nxt = []
                    for i in range(0, len(regs), 2):
                        pa = regs[i] + _lane_shuffle_xor(regs[i], step)
                        pb = regs[i + 1] + _lane_shuffle_xor(regs[i + 1], step)
                        nxt.append(jnp.where(sel, pb, pa))
                    regs = nxt
                    step *= 2
                tot = regs[0]
                if G < L:
                    tot = tot + _lane_shuffle_xor(tot, G)

                vv = tot * (1.0 / H) + EPS
                y = lax.bitcast_convert_type(vv, jnp.int32)
                y = jnp.int32(0x5F3759DF) - (y >> 1)
                r = lax.bitcast_convert_type(y, jnp.float32)
                for _ in range(2):
                    r = r * (1.5 - (0.5 * vv) * (r * r))
                rs = [r[t] for t in range(G)]

                def pass2(j, _):
                    sl = pl.ds(j * L, L)
                    nwj = nw_v[sl]
                    for t in range(G):
                        ob[gb + t, sl] = ob[gb + t, sl] * rs[t] * nwj
                    return 0
                lax.fori_loop(0, NJ, pass2, 0)
                return 0

            lax.fori_loop(0, NGRP, group_body, 0)

            out_start(c, b)

            @pl.when(k < (NCHUNK // 2) - 1)
            def _():
                gather_start(c + 2, b)
        return 0

    lax.fori_loop(0, NCHUNK // 2, outer, 0)
    out_wait(NCHUNK - 2, 0)
    out_wait(NCHUNK - 1, 1)


def kernel(input_ids, token_type_ids, word_emb, type_emb, norm_w):
    ids3 = input_ids.reshape(NW, NCHUNK, C)
    tids2 = token_type_ids.reshape(NW, TPW)
    out = _sc_embed_norm(ids3, tids2, word_emb, type_emb, norm_w)
    return out.reshape(B, S, H)


# speedup vs baseline: 1.3769x; 1.3769x over previous
import functools

import jax
import jax.numpy as jnp
from jax import lax
from jax.experimental import pallas as pl
from jax.experimental.pallas import tpu as pltpu
from jax.experimental.pallas import tpu_sc as plsc

VOCAB = 50265
H = 768
B = 4
S = 8192
EPS = 1e-06

NC, NS, L = 2, 16, 16
NW = NC * NS
NT = B * S
TPW = NT // NW
C = 64
NCHUNK = TPW // C
NJ = H // L
G = 16
NGRP = C // G

_mesh = plsc.VectorSubcoreMesh(core_axis_name="c", subcore_axis_name="s")

_GATHER_DN = lax.GatherDimensionNumbers(
    offset_dims=(), collapsed_slice_dims=(0,), start_index_map=(0,))


def _lane_shuffle_xor(x, sft):
    idx = (lax.iota(jnp.int32, L) ^ sft).reshape(L, 1)
    return lax.gather(x, idx, _GATHER_DN, (1,),
                      mode=lax.GatherScatterMode.PROMISE_IN_BOUNDS)


@functools.partial(
    pl.kernel,
    mesh=_mesh,
    out_type=jax.ShapeDtypeStruct((NT, H), jnp.float32),
    scratch_types=[
        pltpu.VMEM((NCHUNK, C), jnp.int32),
        pltpu.VMEM((TPW,), jnp.int32),
        pltpu.VMEM((TPW + L,), jnp.float32),
        pltpu.VMEM((2, H), jnp.float32),
        pltpu.VMEM((H,), jnp.float32),
        pltpu.VMEM((H,), jnp.float32),
        pltpu.VMEM((C, H), jnp.float32),
        pltpu.VMEM((C, H), jnp.float32),
        pltpu.SemaphoreType.DMA,
        pltpu.SemaphoreType.DMA,
        pltpu.SemaphoreType.DMA,
        pltpu.SemaphoreType.DMA,
    ],
)
def _sc_embed_norm(ids_h, tids_h, we_h, te_h, nw_h, out_h,
                   idx_v, tid_v, tidf_v, te_v, dte_v, nw_v,
                   in0, in1, gs0, gs1, os0, os1):
    wid = lax.axis_index("s") * NC + lax.axis_index("c")
    pltpu.sync_copy(ids_h.at[wid], idx_v)
    pltpu.sync_copy(tids_h.at[wid], tid_v)
    pltpu.sync_copy(te_h, te_v)
    pltpu.sync_copy(nw_h, nw_v)

    ins = (in0, in1)
    gsems = (gs0, gs1)
    osems = (os0, os1)

    for j in range(NJ):
        sl = pl.ds(j * L, L)
        dte_v[sl] = te_v[1, sl] - te_v[0, sl]

    def cvt(k, _):
        sl = pl.ds(k * L, L)
        tidf_v[sl] = tid_v[sl].astype(jnp.float32)
        return 0
    lax.fori_loop(0, TPW // L, cvt, 0)

    def gather_start(c, b):
        pltpu.async_copy(we_h.at[idx_v.at[c]], ins[b], gsems[b])

    def gather_wait(c, b):
        pltpu.make_async_copy(we_h.at[idx_v.at[c]], ins[b], gsems[b]).wait()

    def out_start(c, b):
        dst = out_h.at[pl.ds(pl.multiple_of(wid * TPW + c * C, C), C)]
        pltpu.async_copy(ins[b], dst, osems[b])

    def out_wait(c, b):
        dst = out_h.at[pl.ds(pl.multiple_of(wid * TPW + c * C, C), C)]
        pltpu.make_async_copy(ins[b], dst, osems[b]).wait()

    gather_start(0, 0)

    def outer(k, _):
        for b in range(2):
            c = k * 2 + b
            ib = ins[b]
            ob = ib
            gather_wait(c, b)

            for g in range(NGRP):
                if g == 1:
                    nb = (b + 1) % 2
                    if b == 1:
                        out_wait(c - 1, nb)
                    else:
                        @pl.when(k >= 1)
                        def _():
                            out_wait(c - 1, nb)

                    @pl.when(c + 1 < NCHUNK)
                    def _():
                        gather_start(c + 1, nb)
                gb = g * G
                tfv = tidf_v[pl.ds(c * C + gb, L)]
                tfbs = [jnp.full((L,), tfv[t]) for t in range(G)]

                def pass1(j, accs):
                    sl = pl.ds(j * L, L)
                    te0 = te_v[0, sl]
                    dte = dte_v[sl]
                    out = []
                    for t in range(G):
                        x = ib[gb + t, sl] + (te0 + tfbs[t] * dte)
                        ob[gb + t, sl] = x
                        out.append(accs[t] + x * x)
                    return tuple(out)

                zeros = tuple(jnp.zeros((L,), jnp.float32) for _ in range(G))
                accs = lax.fori_loop(0, NJ, pass1, zeros)

                regs = list(accs)
                step = 1
                lanes = lax.iota(jnp.int32, L)
                while len(regs) > 1:
                    sel = (lanes & step) != 0
                    nxt = []
                    for i in range(0, len(regs), 2):
                        pa = regs[i] + _lane_shuffle_xor(regs[i], step)
                        pb = regs[i + 1] + _lane_shuffle_xor(regs[i + 1], step)
                        nxt.append(jnp.where(sel, pb, pa))
                    regs = nxt
                    step *= 2
                tot = regs[0]
                if G < L:
                    tot = tot + _lane_shuffle_xor(tot, G)

                vv = tot * (1.0 / H) + EPS
                y = lax.bitcast_convert_type(vv, jnp.int32)
                y = jnp.int32(0x5F3759DF) - (y >> 1)
                r = lax.bitcast_convert_type(y, jnp.float32)
                for _ in range(2):
                    r = r * (1.5 - (0.5 * vv) * (r * r))
                rs = [r[t] for t in range(G)]

                def pass2(j, _):
                    sl = pl.ds(j * L, L)
                    nwj = nw_v[sl]
                    for t in range(G):
                        ob[gb + t, sl] = ob[gb + t, sl] * rs[t] * nwj
                    return 0
                lax.fori_loop(0, NJ, pass2, 0)

            out_start(c, b)
        return 0

    lax.fori_loop(0, NCHUNK // 2, outer, 0)
    out_wait(NCHUNK - 1, (NCHUNK - 1) % 2)


def kernel(input_ids, token_type_ids, word_emb, type_emb, norm_w):
    ids3 = input_ids.reshape(NW, NCHUNK, C)
    tids2 = token_type_ids.reshape(NW, TPW)
    out = _sc_embed_norm(ids3, tids2, word_emb, type_emb, norm_w)
    return out.reshape(B, S, H)


# speedup vs baseline: 1.4063x; 1.0213x over previous
import functools

import jax
import jax.numpy as jnp
from jax import lax
from jax.experimental import pallas as pl
from jax.experimental.pallas import tpu as pltpu
from jax.experimental.pallas import tpu_sc as plsc

VOCAB = 50265
H = 768
B = 4
S = 8192
EPS = 1e-06

NC, NS, L = 2, 16, 16
NW = NC * NS
NT = B * S
TPW = NT // NW
C = 64
NCHUNK = TPW // C
NJ = H // L
G = 16
NGRP = C // G

_mesh = plsc.VectorSubcoreMesh(core_axis_name="c", subcore_axis_name="s")

_GATHER_DN = lax.GatherDimensionNumbers(
    offset_dims=(), collapsed_slice_dims=(0,), start_index_map=(0,))


def _lane_shuffle_xor(x, sft):
    idx = (lax.iota(jnp.int32, L) ^ sft).reshape(L, 1)
    return lax.gather(x, idx, _GATHER_DN, (1,),
                      mode=lax.GatherScatterMode.PROMISE_IN_BOUNDS)


@functools.partial(
    pl.kernel,
    mesh=_mesh,
    out_type=jax.ShapeDtypeStruct((NT, H), jnp.float32),
    scratch_types=[
        pltpu.VMEM((NCHUNK, C), jnp.int32),
        pltpu.VMEM((TPW,), jnp.int32),
        pltpu.VMEM((TPW + L,), jnp.float32),
        pltpu.VMEM((2, H), jnp.float32),
        pltpu.VMEM((H,), jnp.float32),
        pltpu.VMEM((H,), jnp.float32),
        pltpu.VMEM((C, H), jnp.float32),
        pltpu.VMEM((C, H), jnp.float32),
        pltpu.SemaphoreType.DMA,
        pltpu.SemaphoreType.DMA,
        pltpu.SemaphoreType.DMA,
        pltpu.SemaphoreType.DMA,
    ],
)
def _sc_embed_norm(ids_h, tids_h, we_h, te_h, nw_h, out_h,
                   idx_v, tid_v, tidf_v, te_v, dte_v, nw_v,
                   in0, in1, gs0, gs1, os0, os1):
    wid = lax.axis_index("s") * NC + lax.axis_index("c")
    ins = (in0, in1)
    gsems = (gs0, gs1)
    osems = (os0, os1)

    cp_idx = pltpu.async_copy(ids_h.at[wid], idx_v, gs0)
    cp_tid = pltpu.async_copy(tids_h.at[wid], tid_v, gs1)
    cp_te = pltpu.async_copy(te_h, te_v, os0)
    cp_nw = pltpu.async_copy(nw_h, nw_v, os1)
    cp_idx.wait()
    pltpu.async_copy(we_h.at[idx_v.at[0]], in0, gs0)
    cp_tid.wait()
    cp_te.wait()
    cp_nw.wait()

    for j in range(NJ):
        sl = pl.ds(j * L, L)
        dte_v[sl] = te_v[1, sl] - te_v[0, sl]

    def cvt(k, _):
        sl = pl.ds(k * L, L)
        tidf_v[sl] = tid_v[sl].astype(jnp.float32)
        return 0
    lax.fori_loop(0, TPW // L, cvt, 0)

    def gather_start(c, b):
        pltpu.async_copy(we_h.at[idx_v.at[c]], ins[b], gsems[b])

    def gather_wait(c, b):
        pltpu.make_async_copy(we_h.at[idx_v.at[c]], ins[b], gsems[b]).wait()

    def out_start(c, b):
        dst = out_h.at[pl.ds(pl.multiple_of(wid * TPW + c * C, C), C)]
        pltpu.async_copy(ins[b], dst, osems[b])

    def out_wait(c, b):
        dst = out_h.at[pl.ds(pl.multiple_of(wid * TPW + c * C, C), C)]
        pltpu.make_async_copy(ins[b], dst, osems[b]).wait()

    def outer(k, _):
        for b in range(2):
            c = k * 2 + b
            ib = ins[b]
            ob = ib
            gather_wait(c, b)

            for g in range(NGRP):
                if g == 1:
                    nb = (b + 1) % 2
                    if b == 1:
                        out_wait(c - 1, nb)
                    else:
                        @pl.when(k >= 1)
                        def _():
                            out_wait(c - 1, nb)

                    @pl.when(c + 1 < NCHUNK)
                    def _():
                        gather_start(c + 1, nb)
                gb = g * G
                tfv = tidf_v[pl.ds(c * C + gb, L)]
                tfbs = [jnp.full((L,), tfv[t]) for t in range(G)]

                def pass1(j, accs):
                    sl = pl.ds(j * L, L)
                    te0 = te_v[0, sl]
                    dte = dte_v[sl]
                    out = []
                    for t in range(G):
                        x = ib[gb + t, sl] + (te0 + tfbs[t] * dte)
                        ob[gb + t, sl] = x
                        out.append(accs[t] + x * x)
                    return tuple(out)

                zeros = tuple(jnp.zeros((L,), jnp.float32) for _ in range(G))
                accs = lax.fori_loop(0, NJ, pass1, zeros)

                regs = list(accs)
                step = 1
                lanes = lax.iota(jnp.int32, L)
                while len(regs) > 1:
                    sel = (lanes & step) != 0
                    nxt = []
                    for i in range(0, len(regs), 2):
                        pa = regs[i] + _lane_shuffle_xor(regs[i], step)
                        pb = regs[i + 1] + _lane_shuffle_xor(regs[i + 1], step)
                        nxt.append(jnp.where(sel, pb, pa))
                    regs = nxt
                    step *= 2
                tot = regs[0]
                if G < L:
                    tot = tot + _lane_shuffle_xor(tot, G)

                vv = tot * (1.0 / H) + EPS
                y = lax.bitcast_convert_type(vv, jnp.int32)
                y = jnp.int32(0x5F3759DF) - (y >> 1)
                r = lax.bitcast_convert_type(y, jnp.float32)
                for _ in range(2):
                    r = r * (1.5 - (0.5 * vv) * (r * r))
                rs = [r[t] for t in range(G)]

                def pass2(j, _):
                    sl = pl.ds(j * L, L)
                    nwj = nw_v[sl]
                    for t in range(G):
                        ob[gb + t, sl] = ob[gb + t, sl] * rs[t] * nwj
                    return 0
                lax.fori_loop(0, NJ, pass2, 0)

            out_start(c, b)
        return 0

    lax.fori_loop(0, NCHUNK // 2, outer, 0)
    out_wait(NCHUNK - 1, (NCHUNK - 1) % 2)


def kernel(input_ids, token_type_ids, word_emb, type_emb, norm_w):
    ids3 = input_ids.reshape(NW, NCHUNK, C)
    tids2 = token_type_ids.reshape(NW, TPW)
    out = _sc_embed_norm(ids3, tids2, word_emb, type_emb, norm_w)
    return out.reshape(B, S, H)
